# baseline (speedup 1.0000x reference)
import functools

import jax
import jax.numpy as jnp
from jax import lax
from jax.experimental import pallas as pl
from jax.experimental.pallas import tpu as pltpu
from jax.experimental.pallas import tpu_sc as plsc

_B, _T, _S, _D = 8, 16, 576, 768
_NC, _NS = 2, 16
_NW = _NC * _NS
_CR = 48
_CPF = _S // _CR
_NCH = _B * _T * _CPF
_CPW = _NCH // _NW


def _sc_body(x_hbm, p32_hbm, tok_hbm, out_hbm, tokbuf, stag, pvec, sem):
    w = lax.axis_index("s") * _NC + lax.axis_index("c")

    pltpu.sync_copy(p32_hbm, pvec)

    pltpu.sync_copy(tok_hbm, tokbuf)

    for i in range(_CPW):
        g = w + _NW * i
        f = g // _CPF
        c = g - f * _CPF
        b = f // _T
        t = f - b * _T
        pb = pvec[pl.ds(b, 16)][0]
        masked = t < pb
        dst = out_hbm.at[b, t, pl.ds(c * _CR, _CR)]

        if i >= 2:
            pltpu.make_async_copy(x_hbm.at[0, 0, pl.ds(0, _CR)], stag.at[i % 2], sem).wait()

        @pl.when(masked)
        def _():
            pltpu.async_copy(tokbuf, dst, sem)

        @pl.when(jnp.logical_not(masked))
        def _():
            pltpu.sync_copy(x_hbm.at[b, t, pl.ds(c * _CR, _CR)], stag.at[i % 2])
            pltpu.async_copy(stag.at[i % 2], dst, sem)

    for i in range(2):
        pltpu.make_async_copy(x_hbm.at[0, 0, pl.ds(0, _CR)], stag.at[i], sem).wait()


@functools.partial(jax.jit, static_argnums=())
def _sc_call(x, p32, tok):
    fn = pl.kernel(
        _sc_body,
        out_type=jax.ShapeDtypeStruct((_B, _T, _S, _D), jnp.float32),
        mesh=plsc.VectorSubcoreMesh(core_axis_name="c", subcore_axis_name="s"),
        scratch_types=[
            pltpu.VMEM((_CR, _D), jnp.float32),
            pltpu.VMEM((2, _CR, _D), jnp.float32),
            pltpu.VMEM((32,), jnp.int32),
            pltpu.SemaphoreType.DMA,
        ],
    )
    return fn(x, p32, tok)


def kernel(x, prefix_len, dropout_mask_token):
    p32 = jnp.zeros((32,), jnp.int32).at[:_B].set(prefix_len)
    tokchunk = jnp.broadcast_to(dropout_mask_token[None, :], (_CR, _D))
    return _sc_call(x, p32, tokchunk)


# speedup vs baseline: 1.0021x; 1.0021x over previous
import functools

import jax
import jax.numpy as jnp
from jax import lax
from jax.experimental import pallas as pl
from jax.experimental.pallas import tpu as pltpu
from jax.experimental.pallas import tpu_sc as plsc

_B, _T, _S, _D = 8, 16, 576, 768
_NC, _NS = 2, 16
_NW = _NC * _NS
_CR = 32
_CPF = _S // _CR
_NCH = _B * _T * _CPF
_CPW = _NCH // _NW
_NSL = 4
_AHEAD = 2


def _sc_body(x_hbm, p32_hbm, tok_hbm, out_hbm, tokbuf, stag, pvec, sem_g, sem_s):
    w = lax.axis_index("s") * _NC + lax.axis_index("c")

    pltpu.sync_copy(p32_hbm, pvec)
    pltpu.sync_copy(tok_hbm, tokbuf)

    def params(i):
        g = w + _NW * i
        f = g // _CPF
        c = g - f * _CPF
        b = f // _T
        t = f - b * _T
        pb = pvec[pl.ds(b, 16)][0]
        return b, t, c, t < pb

    def gather(i, prm):
        b, t, c, masked = prm

        @pl.when(jnp.logical_not(masked))
        def _():
            pltpu.async_copy(
                x_hbm.at[b, t, pl.ds(c * _CR, _CR)], stag.at[i % _NSL], sem_g
            )

    prm = [params(i) for i in range(_AHEAD)]
    for i in range(_AHEAD):
        gather(i, prm[i])

    for i in range(_CPW):
        b, t, c, masked = prm[i % _AHEAD]
        if i >= 2:
            pltpu.make_async_copy(x_hbm.at[0, 0, pl.ds(0, _CR)], stag.at[0], sem_s).wait()
        if i + _AHEAD < _CPW:
            nxt = params(i + _AHEAD)
            gather(i + _AHEAD, nxt)
            prm[i % _AHEAD] = nxt
        dst = out_hbm.at[b, t, pl.ds(c * _CR, _CR)]

        @pl.when(masked)
        def _():
            pltpu.async_copy(tokbuf, dst, sem_s)

        @pl.when(jnp.logical_not(masked))
        def _():
            pltpu.make_async_copy(x_hbm.at[0, 0, pl.ds(0, _CR)], stag.at[0], sem_g).wait()
            pltpu.async_copy(stag.at[i % _NSL], dst, sem_s)

    for i in range(2):
        pltpu.make_async_copy(x_hbm.at[0, 0, pl.ds(0, _CR)], stag.at[0], sem_s).wait()


@functools.partial(jax.jit, static_argnums=())
def _sc_call(x, p32, tokchunk):
    fn = pl.kernel(
        _sc_body,
        out_type=jax.ShapeDtypeStruct((_B, _T, _S, _D), jnp.float32),
        mesh=plsc.VectorSubcoreMesh(core_axis_name="c", subcore_axis_name="s"),
        scratch_types=[
            pltpu.VMEM((_CR, _D), jnp.float32),
            pltpu.VMEM((_NSL, _CR, _D), jnp.float32),
            pltpu.VMEM((32,), jnp.int32),
            pltpu.SemaphoreType.DMA,
            pltpu.SemaphoreType.DMA,
        ],
    )
    return fn(x, p32, tokchunk)


def kernel(x, prefix_len, dropout_mask_token):
    p32 = jnp.zeros((32,), jnp.int32).at[:_B].set(prefix_len)
    tokchunk = jnp.broadcast_to(dropout_mask_token[None, :], (_CR, _D))
    return _sc_call(x, p32, tokchunk)
